# Initial kernel scaffold; baseline (speedup 1.0000x reference)
import jax
import jax.numpy as jnp
from jax.experimental import pallas as pl


def kernel(query, reference_points_3d, feats_l0, feats_l1, feats_l2, feats_l3, camera_R, camera_T, camera_K, W_out, b_out, ln_gamma, ln_beta, img_h, img_w):
    raise NotImplementedError("write your pallas kernel here")


# speedup vs baseline: 1954.0692x; 1954.0692x over previous
import functools

import jax
import jax.numpy as jnp
from jax import lax
from jax.experimental import pallas as pl
from jax.experimental.pallas import tpu as pltpu
from jax.experimental.pallas import tpu_sc as plsc

_LEVELS = ((128, 0), (64, 16384), (32, 20480), (16, 21504))
_HW_TOTAL = 21760
_NC = 2
_NS = 16
_NW = _NC * _NS


def _prep_body(pts_ref, cam_ref, img_ref, idx_ref, wgt_ref, cnt_ref, *, V):
    b = pl.program_id(0)
    v = pl.program_id(1)
    x = pts_ref[0, 0]
    y = pts_ref[0, 1]
    z3 = pts_ref[0, 2]

    def c(k):
        return cam_ref[b, v, k]

    Xc = x * c(0) + y * c(1) + z3 * c(2) + c(9)
    Yc = x * c(3) + y * c(4) + z3 * c(5) + c(10)
    Zc = x * c(6) + y * c(7) + z3 * c(8) + c(11)
    fx, fy, cx, cy = c(12), c(13), c(14), c(15)
    zc = jnp.maximum(Zc, 0.1)
    Hf = img_ref[0]
    Wf = img_ref[1]
    u = Xc * fx / zc + cx
    vv = Yc * fy / zc + cy
    u_n = 2.0 * u / (Wf - 1.0) - 1.0
    v_n = 2.0 * vv / (Hf - 1.0) - 1.0
    in_view = (u_n > -1.0) & (u_n < 1.0) & (v_n > -1.0) & (v_n < 1.0) & (Zc > 0.0)
    ivf = in_view.astype(jnp.float32)

    @pl.when(v == 0)
    def _():
        cnt_ref[0] = ivf

    @pl.when(v != 0)
    def _():
        cnt_ref[0] += ivf

    base = (b * V + v) * _HW_TOTAL
    for l, (Wl, off) in enumerate(_LEVELS):
        ix = ((u_n + 1.0) * Wl - 1.0) / 2.0
        iy = ((v_n + 1.0) * Wl - 1.0) / 2.0
        ix0 = jnp.floor(ix)
        iy0 = jnp.floor(iy)
        wx1 = ix - ix0
        wx0 = 1.0 - wx1
        wy1 = iy - iy0
        wy0 = 1.0 - wy1
        corners = ((ix0, iy0, wx0 * wy0), (ix0 + 1.0, iy0, wx1 * wy0),
                   (ix0, iy0 + 1.0, wx0 * wy1), (ix0 + 1.0, iy0 + 1.0, wx1 * wy1))
        for ci, (xq, yq, wq) in enumerate(corners):
            valid = ((xq >= 0.0) & (xq <= Wl - 1.0)
                     & (yq >= 0.0) & (yq <= Wl - 1.0))
            ixc = jnp.clip(xq, 0.0, Wl - 1.0).astype(jnp.int32)
            iyc = jnp.clip(yq, 0.0, Wl - 1.0).astype(jnp.int32)
            row = base + off + iyc * Wl + ixc
            w = 0.25 * wq * valid.astype(jnp.float32) * ivf
            jl = l * 4 + ci
            idx_ref[jl, 0] = row
            wgt_ref[jl, 0] = w


def _prep(pts, cam, img, B, V, nqr):
    J = V * 16
    return pl.pallas_call(
        functools.partial(_prep_body, V=V),
        grid=(B, V),
        in_specs=[
            pl.BlockSpec((1, 3, nqr, 128), lambda b, v: (b, 0, 0, 0)),
            pl.BlockSpec(memory_space=pltpu.SMEM),
            pl.BlockSpec(memory_space=pltpu.SMEM),
        ],
        out_specs=(
            pl.BlockSpec((16, 1, nqr, 128), lambda b, v: (v, b, 0, 0)),
            pl.BlockSpec((16, 1, nqr, 128), lambda b, v: (v, b, 0, 0)),
            pl.BlockSpec((1, nqr, 128), lambda b, v: (b, 0, 0)),
        ),
        out_shape=(
            jax.ShapeDtypeStruct((J, B, nqr, 128), jnp.int32),
            jax.ShapeDtypeStruct((J, B, nqr, 128), jnp.float32),
            jax.ShapeDtypeStruct((B, nqr, 128), jnp.float32),
        ),
    )(pts, cam, img)


def _bcast_lane(vec, jj):
    idx = jnp.full((16, 1), jj, jnp.int32)
    return lax.gather(
        vec, idx,
        dimension_numbers=lax.GatherDimensionNumbers(
            offset_dims=(), collapsed_slice_dims=(0,), start_index_map=(0,)),
        slice_sizes=(1,),
        mode=lax.GatherScatterMode.PROMISE_IN_BOUNDS)


def _sc_gather_accumulate(table, idx, wgt):
    nq_tot, J = idx.shape
    C = table.shape[1]
    qpw = nq_tot // _NW
    OB = 8
    nchunk = C // 16
    mesh = plsc.VectorSubcoreMesh(core_axis_name="c", subcore_axis_name="s")

    @functools.partial(
        pl.kernel,
        out_type=jax.ShapeDtypeStruct((nq_tot, C), jnp.float32),
        mesh=mesh,
        scratch_types=[
            pltpu.VMEM((qpw, J), jnp.int32),
            pltpu.VMEM((qpw, J), jnp.float32),
            pltpu.VMEM((J, C), jnp.float32),
            pltpu.VMEM((J, C), jnp.float32),
            pltpu.VMEM((OB, C), jnp.float32),
            pltpu.SemaphoreType.DMA,
            pltpu.SemaphoreType.DMA,
        ],
    )
    def run(table_hbm, idx_hbm, wgt_hbm, out_hbm, idx_v, w_v, buf0, buf1, ost,
            sem0, sem1):
        wid = lax.axis_index("s") * _NC + lax.axis_index("c")
        base = wid * qpw
        pltpu.sync_copy(idx_hbm.at[pl.ds(base, qpw)], idx_v)
        pltpu.sync_copy(wgt_hbm.at[pl.ds(base, qpw)], w_v)

        def start(q, buf, sem):
            pltpu.make_async_copy(table_hbm.at[idx_v.at[q]], buf, sem).start()

        def wait(buf, sem):
            pltpu.make_async_copy(table_hbm.at[pl.ds(0, J)], buf, sem).wait()

        def accum(q, buf):
            accs = tuple(jnp.zeros((16,), jnp.float32) for _ in range(nchunk))
            for tj in range(J // 16):
                wrow = w_v[q, pl.ds(16 * tj, 16)]

                def jjbody(jj, accs, tj=tj, wrow=wrow):
                    wb = _bcast_lane(wrow, jj)
                    j = 16 * tj + jj
                    return tuple(a + wb * buf[j, pl.ds(16 * t, 16)]
                                 for t, a in enumerate(accs))

                accs = lax.fori_loop(0, 16, jjbody, accs, unroll=4)
            qm = lax.rem(q, OB)
            for t in range(nchunk):
                ost[qm, pl.ds(16 * t, 16)] = accs[t]

        start(0, buf0, sem0)
        start(1, buf1, sem1)

        def qbody(i, carry):
            q0 = 2 * i
            wait(buf0, sem0)
            accum(q0, buf0)

            @pl.when(q0 + 2 < qpw)
            def _():
                start(q0 + 2, buf0, sem0)

            wait(buf1, sem1)
            accum(q0 + 1, buf1)

            @pl.when(q0 + 3 < qpw)
            def _():
                start(q0 + 3, buf1, sem1)

            @pl.when(lax.rem(q0, OB) == OB - 2)
            def _():
                pltpu.sync_copy(ost, out_hbm.at[pl.ds(base + (q0 // OB) * OB, OB)])

            return carry

        lax.fori_loop(0, qpw // 2, qbody, 0)

    return run(table, idx, wgt)


def _finish_body(q_ref, acc_ref, cnt_ref, wt_ref, b_ref, g_ref, be_ref, o_ref):
    f = acc_ref[...] / jnp.maximum(cnt_ref[...], 1.0)
    o = q_ref[...] + jnp.dot(f, wt_ref[...],
                             preferred_element_type=jnp.float32) + b_ref[...]
    mu = jnp.mean(o, axis=-1, keepdims=True)
    d = o - mu
    var = jnp.mean(d * d, axis=-1, keepdims=True)
    o_ref[...] = d * lax.rsqrt(var + 1e-5) * g_ref[...] + be_ref[...]


def _finish(q2, acc, cnt2, w_t, b2, g2, be2, blk=512):
    n, C = q2.shape
    return pl.pallas_call(
        _finish_body,
        grid=(n // blk,),
        in_specs=[
            pl.BlockSpec((blk, C), lambda i: (i, 0)),
            pl.BlockSpec((blk, C), lambda i: (i, 0)),
            pl.BlockSpec((blk, 1), lambda i: (i, 0)),
            pl.BlockSpec((C, C), lambda i: (0, 0)),
            pl.BlockSpec((1, C), lambda i: (0, 0)),
            pl.BlockSpec((1, C), lambda i: (0, 0)),
            pl.BlockSpec((1, C), lambda i: (0, 0)),
        ],
        out_specs=pl.BlockSpec((blk, C), lambda i: (i, 0)),
        out_shape=jax.ShapeDtypeStruct((n, C), jnp.float32),
    )(q2, acc, cnt2, w_t, b2, g2, be2)


def kernel(query, reference_points_3d, feats_l0, feats_l1, feats_l2, feats_l3,
           camera_R, camera_T, camera_K, W_out, b_out, ln_gamma, ln_beta,
           img_h, img_w):
    B, Nq, C = query.shape
    V = camera_R.shape[1]
    nqr = Nq // 128
    J = V * 16

    parts = []
    for f in (feats_l0, feats_l1, feats_l2, feats_l3):
        H, W = f.shape[3], f.shape[4]
        parts.append(f.transpose(0, 1, 3, 4, 2).reshape(B, V, H * W, C))
    table = jnp.concatenate(parts, axis=2).reshape(B * V * _HW_TOTAL, C)

    pts = reference_points_3d.transpose(0, 2, 1).reshape(B, 3, nqr, 128)
    Rr = camera_R.reshape(B, V, 9)
    Kf = jnp.stack([camera_K[..., 0, 0], camera_K[..., 1, 1],
                    camera_K[..., 0, 2], camera_K[..., 1, 2]], axis=-1)
    cam = jnp.concatenate([Rr, camera_T, Kf], axis=-1)
    img = jnp.stack([jnp.float32(img_h), jnp.float32(img_w)])

    idx_t, wgt_t, cnt = _prep(pts, cam, img, B, V, nqr)
    idx = idx_t.reshape(J, B * Nq).T
    wgt = wgt_t.reshape(J, B * Nq).T

    acc = _sc_gather_accumulate(table, idx, wgt)

    out = _finish(query.reshape(B * Nq, C), acc, cnt.reshape(B * Nq, 1),
                  W_out.T, b_out.reshape(1, C), ln_gamma.reshape(1, C),
                  ln_beta.reshape(1, C))
    return out.reshape(B, Nq, C)
